# baseline (speedup 1.0000x reference)
import functools

import jax
import jax.numpy as jnp
from jax import lax
from jax.experimental import pallas as pl
from jax.experimental.pallas import tpu as pltpu
from jax.experimental.pallas import tpu_sc as plsc

B, P, K = 8, 1024, 32
N = B * P
HID = [64, 128, 256]
NC, NS = 2, 16
NW = NC * NS
PW = N // NW


NBLK = 8
BW = P // NBLK


def _knn(x_ref, idx_ref):
    b = pl.program_id(0)
    xb = x_ref[...]
    sq = jnp.sum(xb * xb, axis=1, keepdims=True)
    gram = lax.dot_general(xb, xb, (((1,), (1,)), ((), ())),
                           preferred_element_type=jnp.float32)
    d2 = (sq + sq.T) - 2.0 * gram
    d23 = d2.reshape(P, NBLK, BW)
    M = jnp.min(d23, axis=2)
    iota_b = lax.broadcasted_iota(jnp.int32, (P, NBLK), 1)
    iota_w = lax.broadcasted_iota(jnp.int32, (P, BW), 1)
    iota_k = lax.broadcasted_iota(jnp.int32, (P, K), 1)
    inf = jnp.float32(jnp.inf)

    def step(k, carry):
        d23, M, acc = carry
        m = jnp.min(M, axis=1, keepdims=True)
        candb = jnp.where(M == m, iota_b, NBLK)
        bsel = jnp.min(candb, axis=1, keepdims=True)
        w = d23[:, NBLK - 1, :]
        for bb in range(NBLK - 2, -1, -1):
            w = jnp.where(bsel == bb, d23[:, bb, :], w)
        candw = jnp.where(w == m, iota_w, BW)
        jloc = jnp.min(candw, axis=1, keepdims=True)
        wp = jnp.where(candw == jloc, inf, w)
        Mwin = jnp.min(wp, axis=1, keepdims=True)
        M = jnp.where(candb == bsel, Mwin, M)
        cols = [jnp.where(bsel == bb, wp, d23[:, bb, :])
                for bb in range(NBLK)]
        d23 = jnp.stack(cols, axis=1)
        acc = jnp.where(iota_k == k, bsel * BW + jloc + b * P, acc)
        return d23, M, acc

    _, _, acc = lax.fori_loop(
        0, K, step, (d23, M, jnp.zeros((P, K), jnp.int32)))
    idx_ref[...] = acc


def _topk(x):
    d = x.shape[1]
    return pl.pallas_call(
        _knn,
        grid=(B,),
        in_specs=[pl.BlockSpec((P, d), lambda b: (b, 0))],
        out_specs=pl.BlockSpec((P, K), lambda b: (b, 0)),
        out_shape=jax.ShapeDtypeStruct((N, K), jnp.int32),
    )(x)


def _knn_zc(x_ref, wb_ref, wc_ref, bias_ref, idx_ref, z_ref, c_ref):
    _knn(x_ref, idx_ref)
    xb = x_ref[...]
    z_ref[...] = lax.dot_general(xb, wb_ref[...], (((1,), (0,)), ((), ())),
                                 preferred_element_type=jnp.float32,
                                 precision=lax.Precision.HIGHEST)
    c_ref[...] = lax.dot_general(xb, wc_ref[...], (((1,), (0,)), ((), ())),
                                 preferred_element_type=jnp.float32,
                                 precision=lax.Precision.HIGHEST) + bias_ref[...]


def _topk_zc(x, wb, wc, bias):
    d = x.shape[1]
    fo = wb.shape[1]
    return pl.pallas_call(
        _knn_zc,
        grid=(B,),
        in_specs=[pl.BlockSpec((P, d), lambda b: (b, 0)),
                  pl.BlockSpec((d, fo), lambda b: (0, 0)),
                  pl.BlockSpec((d, fo), lambda b: (0, 0)),
                  pl.BlockSpec((1, fo), lambda b: (0, 0))],
        out_specs=[pl.BlockSpec((P, K), lambda b: (b, 0)),
                   pl.BlockSpec((P, fo), lambda b: (b, 0)),
                   pl.BlockSpec((P, fo), lambda b: (b, 0))],
        out_shape=[jax.ShapeDtypeStruct((N, K), jnp.int32),
                   jax.ShapeDtypeStruct((N, fo), jnp.float32),
                   jax.ShapeDtypeStruct((N, fo), jnp.float32)],
    )(x, wb, wc, bias)


def _edge_body(x_ref, xj_ref, w_ref, bias_ref, h_ref, *, dpad, stride, fo):
    x = x_ref[...]
    w = w_ref[...]
    acc = jnp.full((P, fo), -jnp.inf, jnp.float32)
    for k in range(K):
        xj = xj_ref[:, k * stride:k * stride + dpad]
        e = jnp.concatenate([x, xj - x], axis=1)
        hk = lax.dot_general(e, w, (((1,), (0,)), ((), ())),
                             preferred_element_type=jnp.float32)
        acc = jnp.maximum(acc, hk)
    h_ref[...] = acc + bias_ref[...]


def _edge_conv(x, xj, w, bias, stride):
    dpad = x.shape[1]
    fo = w.shape[1]
    return pl.pallas_call(
        functools.partial(_edge_body, dpad=dpad, stride=stride, fo=fo),
        grid=(B,),
        in_specs=[pl.BlockSpec((P, dpad), lambda b: (b, 0)),
                  pl.BlockSpec((P, K * stride), lambda b: (b, 0)),
                  pl.BlockSpec((2 * dpad, fo), lambda b: (0, 0)),
                  pl.BlockSpec((1, fo), lambda b: (0, 0))],
        out_specs=pl.BlockSpec((P, fo), lambda b: (b, 0)),
        out_shape=jax.ShapeDtypeStruct((N, fo), jnp.float32),
    )(x, xj, w, bias)


def _bn_body(in_ref, g_ref, beta_ref, o_ref):
    u = jnp.maximum(in_ref[...], 0.0)
    mu = jnp.mean(u, axis=0, keepdims=True)
    var = jnp.mean((u - mu) ** 2, axis=0, keepdims=True)
    o_ref[...] = g_ref[...] * (u - mu) * lax.rsqrt(var + 1e-5) + beta_ref[...]


def _bn_relu(h, g, beta):
    fo = h.shape[1]
    return pl.pallas_call(
        _bn_body,
        in_specs=[pl.BlockSpec((N, fo), lambda: (0, 0)),
                  pl.BlockSpec((1, fo), lambda: (0, 0)),
                  pl.BlockSpec((1, fo), lambda: (0, 0))],
        out_specs=pl.BlockSpec((N, fo), lambda: (0, 0)),
        out_shape=jax.ShapeDtypeStruct((N, fo), jnp.float32),
    )(h, g.reshape(1, fo), beta.reshape(1, fo))


def _final_body(m_ref, c_ref, y_ref, h_ref, l_ref):
    h = m_ref[...] + c_ref[...]
    h_ref[...] = h
    d = h - y_ref[...]
    part = jnp.sum(d * d, axis=(0, 1), keepdims=True) / jnp.float32(N * HID[2])

    @pl.when(pl.program_id(0) == 0)
    def _():
        l_ref[...] = part

    @pl.when(pl.program_id(0) > 0)
    def _():
        l_ref[...] += part


def _final(m, c, y):
    fo = HID[2]
    h, loss = pl.pallas_call(
        _final_body,
        grid=(B,),
        in_specs=[pl.BlockSpec((P, fo), lambda b: (b, 0)),
                  pl.BlockSpec((P, fo), lambda b: (b, 0)),
                  pl.BlockSpec((P, fo), lambda b: (b, 0))],
        out_specs=[pl.BlockSpec((P, fo), lambda b: (b, 0)),
                   pl.BlockSpec((1, 1), lambda b: (0, 0))],
        out_shape=[jax.ShapeDtypeStruct((N, fo), jnp.float32),
                   jax.ShapeDtypeStruct((1, 1), jnp.float32)],
    )(m, c, y)
    return h, loss.reshape(())


def _make_gather(dpad):
    EW = PW * K
    GRP = 128
    SUBS = GRP // 16
    mesh = plsc.VectorSubcoreMesh(core_axis_name="c", subcore_axis_name="s",
                                  num_cores=NC, num_subcores=NS)

    def body(table_hbm, idx_hbm, out_hbm, tab_v, idx_v, ob):
        wid = lax.axis_index("s") * NC + lax.axis_index("c")
        cloud = wid // (NW // B)
        ebase = wid * EW
        pltpu.sync_copy(table_hbm.at[pl.ds(cloud * P * dpad, P * dpad)], tab_v)
        pltpu.sync_copy(idx_hbm.at[pl.ds(ebase, EW)], idx_v)
        iota16 = lax.broadcasted_iota(jnp.int32, (16,), 0)
        cbase = cloud * P

        def group(g, _):
            for sub in range(SUBS):
                jd = (idx_v[pl.ds(g * GRP + sub * 16, 16)] - cbase) * dpad
                rd = (iota16 + sub * 16) * dpad
                for f in range(dpad):
                    vals = plsc.load_gather(tab_v, [jd + f])
                    plsc.store_scatter(ob, [rd + f], vals)
            pltpu.sync_copy(
                ob, out_hbm.at[pl.ds((ebase + g * GRP) * dpad, GRP * dpad)])
            return 0

        lax.fori_loop(0, EW // GRP, group, 0)

    return pl.kernel(
        body,
        out_type=jax.ShapeDtypeStruct((N * K * dpad,), jnp.float32),
        mesh=mesh,
        compiler_params=pltpu.CompilerParams(needs_layout_passes=False),
        scratch_types=[pltpu.VMEM((P * dpad,), jnp.float32),
                       pltpu.VMEM((EW,), jnp.int32),
                       pltpu.VMEM((GRP * dpad,), jnp.float32)])


def _make_stream_gather(d, cpts):
    rows = cpts * K
    nchunks = PW // cpts
    npairs = nchunks // 2
    mesh = plsc.VectorSubcoreMesh(core_axis_name="c", subcore_axis_name="s",
                                  num_cores=NC, num_subcores=NS)

    def body(table_hbm, idx_hbm, out_hbm, idx_v, buf0, buf1, sem0, sem1):
        wid = lax.axis_index("s") * NC + lax.axis_index("c")
        ebase = wid * (PW * K)
        pltpu.sync_copy(idx_hbm.at[pl.ds(ebase, PW * K)], idx_v)

        def start(c, buf, sem):
            pltpu.async_copy(
                table_hbm.at[idx_v.at[pl.ds(c * rows, rows)]], buf, sem)

        def wait(buf, sem):
            pltpu.make_async_copy(
                table_hbm.at[pl.ds(0, rows)], buf, sem).wait()

        start(0, buf0, sem0)

        def pair(i, _):
            start(2 * i + 1, buf1, sem1)
            wait(buf0, sem0)
            pltpu.sync_copy(
                buf0, out_hbm.at[pl.ds(ebase + (2 * i) * rows, rows)])

            @pl.when(i < npairs - 1)
            def _():
                start(2 * i + 2, buf0, sem0)

            wait(buf1, sem1)
            pltpu.sync_copy(
                buf1, out_hbm.at[pl.ds(ebase + (2 * i + 1) * rows, rows)])
            return 0

        lax.fori_loop(0, npairs, pair, 0)

    return pl.kernel(
        body,
        out_type=jax.ShapeDtypeStruct((N * K, d), jnp.float32),
        mesh=mesh,
        scratch_types=[pltpu.VMEM((PW * K,), jnp.int32),
                       pltpu.VMEM((rows, d), jnp.float32),
                       pltpu.VMEM((rows, d), jnp.float32),
                       pltpu.SemaphoreType.DMA,
                       pltpu.SemaphoreType.DMA])


def _make_gathermax(d, cpts):
    rows = cpts * K
    nchunks = PW // cpts
    mesh = plsc.VectorSubcoreMesh(core_axis_name="c", subcore_axis_name="s",
                                  num_cores=NC, num_subcores=NS)

    npairs = nchunks // 2

    def body(z_hbm, idx_hbm, out_hbm, idx_v, buf0, buf1, ob, sem0, sem1):
        wid = lax.axis_index("s") * NC + lax.axis_index("c")
        ebase = wid * (PW * K)
        pbase = wid * PW
        pltpu.sync_copy(idx_hbm.at[pl.ds(ebase, PW * K)], idx_v)

        def start(c, buf, sem):
            pltpu.async_copy(
                z_hbm.at[idx_v.at[pl.ds(c * rows, rows)]], buf, sem)

        def wait(buf, sem):
            pltpu.make_async_copy(z_hbm.at[pl.ds(0, rows)], buf, sem).wait()

        def reduce_write(c, buf):
            for p in range(cpts):
                for f in range(d // 16):
                    s = pl.ds(f * 16, 16)
                    acc = buf[p * K, s]
                    for r in range(1, K):
                        acc = jnp.maximum(acc, buf[p * K + r, s])
                    ob[p, s] = acc
            pltpu.sync_copy(ob, out_hbm.at[pl.ds(pbase + c * cpts, cpts)])

        start(0, buf0, sem0)

        def pair(i, _):
            start(2 * i + 1, buf1, sem1)
            wait(buf0, sem0)
            reduce_write(2 * i, buf0)

            @pl.when(i < npairs - 1)
            def _():
                start(2 * i + 2, buf0, sem0)

            wait(buf1, sem1)
            reduce_write(2 * i + 1, buf1)
            return 0

        lax.fori_loop(0, npairs, pair, 0)

    return pl.kernel(
        body,
        out_type=jax.ShapeDtypeStruct((N, d), jnp.float32),
        mesh=mesh,
        scratch_types=[pltpu.VMEM((PW * K,), jnp.int32),
                       pltpu.VMEM((rows, d), jnp.float32),
                       pltpu.VMEM((rows, d), jnp.float32),
                       pltpu.VMEM((cpts, d), jnp.float32),
                       pltpu.SemaphoreType.DMA,
                       pltpu.SemaphoreType.DMA])


def kernel(x, batch, y, W0, b0, W1, b1, W2, b2, g0, beta0, g1, beta1):
    d0 = x.shape[1]
    dp = 16
    xp = jnp.pad(x, ((0, 0), (0, dp - d0)))
    w0p = jnp.zeros((2 * dp, HID[0]), jnp.float32)
    w0p = w0p.at[:d0].set(W0[:d0]).at[dp:dp + d0].set(W0[d0:])

    idx1 = _topk(xp)
    xj1 = _make_gather(dp)(xp.reshape(-1), idx1.reshape(-1))
    h = _edge_conv(xp, xj1.reshape(N, K * dp), w0p, b0.reshape(1, -1),
                   stride=dp)
    h = _bn_relu(h, g0, beta0)

    idx2 = _topk(h)
    hp = jnp.pad(h, ((0, 0), (0, 128 - HID[0])))
    xj2 = _make_stream_gather(128, 8)(hp, idx2.reshape(-1))
    h = _edge_conv(h, xj2.reshape(N, K * 128), W1, b1.reshape(1, -1),
                   stride=128)
    h = _bn_relu(h, g1, beta1)

    wb2 = W2[HID[1]:]
    wc2 = W2[:HID[1]] - wb2
    idx3, z, c = _topk_zc(h, wb2, wc2, b2.reshape(1, -1))
    m = _make_gathermax(HID[2], 2)(z, idx3.reshape(-1))
    return _final(m, c, y)


# speedup vs baseline: 4.7440x; 4.7440x over previous
import functools

import jax
import jax.numpy as jnp
from jax import lax
from jax.experimental import pallas as pl
from jax.experimental.pallas import tpu as pltpu
from jax.experimental.pallas import tpu_sc as plsc

B, P, K = 8, 1024, 32
N = B * P
HID = [64, 128, 256]
NC, NS = 2, 16
NW = NC * NS
PW = N // NW


def _knn(x_ref, idx_ref):
    b = pl.program_id(0)
    xb = x_ref[...]
    sq = jnp.sum(xb * xb, axis=1, keepdims=True)
    gram = lax.dot_general(xb, xb, (((1,), (1,)), ((), ())),
                           preferred_element_type=jnp.float32)
    d2 = (sq + sq.T) - 2.0 * gram
    iota_p = lax.broadcasted_iota(jnp.int32, (P, P), 1)
    iota_k = lax.broadcasted_iota(jnp.int32, (P, K), 1)

    def step(k, carry):
        dist, acc = carry
        m = jnp.min(dist, axis=1, keepdims=True)
        j = jnp.min(jnp.where(dist == m, iota_p, P), axis=1,
                    keepdims=True)
        dist = jnp.where(iota_p == j, jnp.float32(jnp.inf), dist)
        acc = jnp.where(iota_k == k, j + b * P, acc)
        return dist, acc

    _, acc = lax.fori_loop(0, K, step, (d2, jnp.zeros((P, K), jnp.int32)))
    idx_ref[...] = acc


def _topk(x):
    d = x.shape[1]
    return pl.pallas_call(
        _knn,
        grid=(B,),
        in_specs=[pl.BlockSpec((P, d), lambda b: (b, 0))],
        out_specs=pl.BlockSpec((P, K), lambda b: (b, 0)),
        out_shape=jax.ShapeDtypeStruct((N, K), jnp.int32),
    )(x)


def _knn_zc(x_ref, wb_ref, wc_ref, bias_ref, idx_ref, z_ref, c_ref):
    _knn(x_ref, idx_ref)
    xb = x_ref[...]
    z = lax.dot_general(xb, wb_ref[...], (((1,), (0,)), ((), ())),
                        preferred_element_type=jnp.float32,
                        precision=lax.Precision.HIGHEST)
    fo = z.shape[1]
    lo = lax.bitcast_convert_type(z[:, :fo // 2].astype(jnp.bfloat16),
                                  jnp.uint16).astype(jnp.uint32)
    hi = lax.bitcast_convert_type(z[:, fo // 2:].astype(jnp.bfloat16),
                                  jnp.uint16).astype(jnp.uint32)
    z_ref[...] = lax.bitcast_convert_type((hi << 16) | lo, jnp.float32)
    c_ref[...] = lax.dot_general(xb, wc_ref[...], (((1,), (0,)), ((), ())),
                                 preferred_element_type=jnp.float32,
                                 precision=lax.Precision.HIGHEST) + bias_ref[...]


def _topk_zc(x, wb, wc, bias):
    d = x.shape[1]
    fo = wb.shape[1]
    return pl.pallas_call(
        _knn_zc,
        grid=(B,),
        in_specs=[pl.BlockSpec((P, d), lambda b: (b, 0)),
                  pl.BlockSpec((d, fo), lambda b: (0, 0)),
                  pl.BlockSpec((d, fo), lambda b: (0, 0)),
                  pl.BlockSpec((1, fo), lambda b: (0, 0))],
        out_specs=[pl.BlockSpec((P, K), lambda b: (b, 0)),
                   pl.BlockSpec((P, fo // 2), lambda b: (b, 0)),
                   pl.BlockSpec((P, fo), lambda b: (b, 0))],
        out_shape=[jax.ShapeDtypeStruct((N, K), jnp.int32),
                   jax.ShapeDtypeStruct((N, fo // 2), jnp.float32),
                   jax.ShapeDtypeStruct((N, fo), jnp.float32)],
    )(x, wb, wc, bias)


def _edge_body(x_ref, xj_ref, w_ref, bias_ref, h_ref, *, dpad, stride, fo):
    x = x_ref[...]
    w = w_ref[...]
    acc = jnp.full((P, fo), -jnp.inf, jnp.float32)
    for k in range(K):
        xj = xj_ref[:, k * stride:k * stride + dpad]
        e = jnp.concatenate([x, xj - x], axis=1)
        hk = lax.dot_general(e, w, (((1,), (0,)), ((), ())),
                             preferred_element_type=jnp.float32)
        acc = jnp.maximum(acc, hk)
    h_ref[...] = acc + bias_ref[...]


def _edge_conv(x, xj, w, bias, stride):
    dpad = x.shape[1]
    fo = w.shape[1]
    return pl.pallas_call(
        functools.partial(_edge_body, dpad=dpad, stride=stride, fo=fo),
        grid=(B,),
        in_specs=[pl.BlockSpec((P, dpad), lambda b: (b, 0)),
                  pl.BlockSpec((P, K * stride), lambda b: (b, 0)),
                  pl.BlockSpec((2 * dpad, fo), lambda b: (0, 0)),
                  pl.BlockSpec((1, fo), lambda b: (0, 0))],
        out_specs=pl.BlockSpec((P, fo), lambda b: (b, 0)),
        out_shape=jax.ShapeDtypeStruct((N, fo), jnp.float32),
    )(x, xj, w, bias)


def _bn_body(in_ref, g_ref, beta_ref, o_ref):
    u = jnp.maximum(in_ref[...], 0.0)
    mu = jnp.mean(u, axis=0, keepdims=True)
    var = jnp.mean((u - mu) ** 2, axis=0, keepdims=True)
    o_ref[...] = g_ref[...] * (u - mu) * lax.rsqrt(var + 1e-5) + beta_ref[...]


def _bn_relu(h, g, beta):
    fo = h.shape[1]
    return pl.pallas_call(
        _bn_body,
        in_specs=[pl.BlockSpec((N, fo), lambda: (0, 0)),
                  pl.BlockSpec((1, fo), lambda: (0, 0)),
                  pl.BlockSpec((1, fo), lambda: (0, 0))],
        out_specs=pl.BlockSpec((N, fo), lambda: (0, 0)),
        out_shape=jax.ShapeDtypeStruct((N, fo), jnp.float32),
    )(h, g.reshape(1, fo), beta.reshape(1, fo))


def _final_body(m_ref, c_ref, y_ref, h_ref, l_ref):
    h = m_ref[...] + c_ref[...]
    h_ref[...] = h
    d = h - y_ref[...]
    part = jnp.sum(d * d, axis=(0, 1), keepdims=True) / jnp.float32(N * HID[2])

    @pl.when(pl.program_id(0) == 0)
    def _():
        l_ref[...] = part

    @pl.when(pl.program_id(0) > 0)
    def _():
        l_ref[...] += part


def _final(m, c, y):
    fo = HID[2]
    h, loss = pl.pallas_call(
        _final_body,
        grid=(B,),
        in_specs=[pl.BlockSpec((P, fo), lambda b: (b, 0)),
                  pl.BlockSpec((P, fo), lambda b: (b, 0)),
                  pl.BlockSpec((P, fo), lambda b: (b, 0))],
        out_specs=[pl.BlockSpec((P, fo), lambda b: (b, 0)),
                   pl.BlockSpec((1, 1), lambda b: (0, 0))],
        out_shape=[jax.ShapeDtypeStruct((N, fo), jnp.float32),
                   jax.ShapeDtypeStruct((1, 1), jnp.float32)],
    )(m, c, y)
    return h, loss.reshape(())


def _make_gather(dpad):
    EW = PW * K
    GRP = 128
    SUBS = GRP // 16
    mesh = plsc.VectorSubcoreMesh(core_axis_name="c", subcore_axis_name="s",
                                  num_cores=NC, num_subcores=NS)

    def body(table_hbm, idx_hbm, out_hbm, tab_v, idx_v, ob):
        wid = lax.axis_index("s") * NC + lax.axis_index("c")
        cloud = wid // (NW // B)
        ebase = wid * EW
        pltpu.sync_copy(table_hbm.at[pl.ds(cloud * P * dpad, P * dpad)], tab_v)
        pltpu.sync_copy(idx_hbm.at[pl.ds(ebase, EW)], idx_v)
        iota16 = lax.broadcasted_iota(jnp.int32, (16,), 0)
        cbase = cloud * P

        def group(g, _):
            for sub in range(SUBS):
                jd = (idx_v[pl.ds(g * GRP + sub * 16, 16)] - cbase) * dpad
                rd = (iota16 + sub * 16) * dpad
                for f in range(dpad):
                    vals = plsc.load_gather(tab_v, [jd + f])
                    plsc.store_scatter(ob, [rd + f], vals)
            pltpu.sync_copy(
                ob, out_hbm.at[pl.ds((ebase + g * GRP) * dpad, GRP * dpad)])
            return 0

        lax.fori_loop(0, EW // GRP, group, 0)

    return pl.kernel(
        body,
        out_type=jax.ShapeDtypeStruct((N * K * dpad,), jnp.float32),
        mesh=mesh,
        compiler_params=pltpu.CompilerParams(needs_layout_passes=False),
        scratch_types=[pltpu.VMEM((P * dpad,), jnp.float32),
                       pltpu.VMEM((EW,), jnp.int32),
                       pltpu.VMEM((GRP * dpad,), jnp.float32)])


def _make_stream_gather(d, cpts):
    rows = cpts * K
    nchunks = PW // cpts
    npairs = nchunks // 2
    mesh = plsc.VectorSubcoreMesh(core_axis_name="c", subcore_axis_name="s",
                                  num_cores=NC, num_subcores=NS)

    def body(table_hbm, idx_hbm, out_hbm, idx_v, buf0, buf1, sem0, sem1):
        wid = lax.axis_index("s") * NC + lax.axis_index("c")
        ebase = wid * (PW * K)
        pltpu.sync_copy(idx_hbm.at[pl.ds(ebase, PW * K)], idx_v)

        def start(c, buf, sem):
            pltpu.async_copy(
                table_hbm.at[idx_v.at[pl.ds(c * rows, rows)]], buf, sem)

        def wait(buf, sem):
            pltpu.make_async_copy(
                table_hbm.at[pl.ds(0, rows)], buf, sem).wait()

        start(0, buf0, sem0)

        def pair(i, _):
            start(2 * i + 1, buf1, sem1)
            wait(buf0, sem0)
            pltpu.sync_copy(
                buf0, out_hbm.at[pl.ds(ebase + (2 * i) * rows, rows)])

            @pl.when(i < npairs - 1)
            def _():
                start(2 * i + 2, buf0, sem0)

            wait(buf1, sem1)
            pltpu.sync_copy(
                buf1, out_hbm.at[pl.ds(ebase + (2 * i + 1) * rows, rows)])
            return 0

        lax.fori_loop(0, npairs, pair, 0)

    return pl.kernel(
        body,
        out_type=jax.ShapeDtypeStruct((N * K, d), jnp.float32),
        mesh=mesh,
        scratch_types=[pltpu.VMEM((PW * K,), jnp.int32),
                       pltpu.VMEM((rows, d), jnp.float32),
                       pltpu.VMEM((rows, d), jnp.float32),
                       pltpu.SemaphoreType.DMA,
                       pltpu.SemaphoreType.DMA])


def _make_gathermax(d, cpts):
    pd = d // 2
    rows = cpts * K
    nchunks = PW // cpts
    mesh = plsc.VectorSubcoreMesh(core_axis_name="c", subcore_axis_name="s",
                                  num_cores=NC, num_subcores=NS)

    npairs = nchunks // 2

    def body(z_hbm, idx_hbm, out_hbm, idx_v, buf0, buf1, ob, sem0, sem1):
        wid = lax.axis_index("s") * NC + lax.axis_index("c")
        ebase = wid * (PW * K)
        pbase = wid * PW
        pltpu.sync_copy(idx_hbm.at[pl.ds(ebase, PW * K)], idx_v)

        def start(c, buf, sem):
            pltpu.async_copy(
                z_hbm.at[idx_v.at[pl.ds(c * rows, rows)]], buf, sem)

        def wait(buf, sem):
            pltpu.make_async_copy(z_hbm.at[pl.ds(0, rows)], buf, sem).wait()

        mhi = jnp.full((16,), -65536, jnp.int32)

        def unpack2(v):
            vi = plsc.bitcast(v, jnp.int32)
            flo = plsc.bitcast(vi << 16, jnp.float32)
            fhi = plsc.bitcast(vi & mhi, jnp.float32)
            return flo, fhi

        def reduce_write(c, buf):
            for p in range(cpts):
                for f in range(pd // 16):
                    s = pl.ds(f * 16, 16)
                    alo, ahi = unpack2(buf[p * K, s])
                    for r in range(1, K):
                        flo, fhi = unpack2(buf[p * K + r, s])
                        alo = jnp.maximum(alo, flo)
                        ahi = jnp.maximum(ahi, fhi)
                    ob[p, s] = alo
                    ob[p, pl.ds(pd + f * 16, 16)] = ahi
            pltpu.sync_copy(ob, out_hbm.at[pl.ds(pbase + c * cpts, cpts)])

        start(0, buf0, sem0)

        def pair(i, _):
            start(2 * i + 1, buf1, sem1)
            wait(buf0, sem0)
            reduce_write(2 * i, buf0)

            @pl.when(i < npairs - 1)
            def _():
                start(2 * i + 2, buf0, sem0)

            wait(buf1, sem1)
            reduce_write(2 * i + 1, buf1)
            return 0

        lax.fori_loop(0, npairs, pair, 0)

    return pl.kernel(
        body,
        out_type=jax.ShapeDtypeStruct((N, d), jnp.float32),
        mesh=mesh,
        compiler_params=pltpu.CompilerParams(needs_layout_passes=False),
        scratch_types=[pltpu.VMEM((PW * K,), jnp.int32),
                       pltpu.VMEM((rows, pd), jnp.float32),
                       pltpu.VMEM((rows, pd), jnp.float32),
                       pltpu.VMEM((cpts, d), jnp.float32),
                       pltpu.SemaphoreType.DMA,
                       pltpu.SemaphoreType.DMA])


def kernel(x, batch, y, W0, b0, W1, b1, W2, b2, g0, beta0, g1, beta1):
    d0 = x.shape[1]
    dp = 16
    xp = jnp.pad(x, ((0, 0), (0, dp - d0)))
    w0p = jnp.zeros((2 * dp, HID[0]), jnp.float32)
    w0p = w0p.at[:d0].set(W0[:d0]).at[dp:dp + d0].set(W0[d0:])

    idx1 = _topk(xp)
    xj1 = _make_gather(dp)(xp.reshape(-1), idx1.reshape(-1))
    h = _edge_conv(xp, xj1.reshape(N, K * dp), w0p, b0.reshape(1, -1),
                   stride=dp)
    h = _bn_relu(h, g0, beta0)

    idx2 = _topk(h)
    hp = jnp.pad(h, ((0, 0), (0, 128 - HID[0])))
    xj2 = _make_stream_gather(128, 8)(hp, idx2.reshape(-1))
    h = _edge_conv(h, xj2.reshape(N, K * 128), W1, b1.reshape(1, -1),
                   stride=128)
    h = _bn_relu(h, g1, beta1)

    wb2 = W2[HID[1]:]
    wc2 = W2[:HID[1]] - wb2
    idx3, z, c = _topk_zc(h, wb2, wc2, b2.reshape(1, -1))
    m = _make_gathermax(HID[2], 2)(z, idx3.reshape(-1))
    return _final(m, c, y)
